# baseline (device time: 37076 ns/iter reference)
import jax
import jax.numpy as jnp
from jax import lax
from jax.experimental import pallas as pl
from jax.experimental.pallas import tpu as pltpu

N_DEV = 16
N_IDX = 1024
D = 512
V_PER = 4096
CHUNK = N_IDX // N_DEV


def kernel(table, idx):
    idx2 = idx.reshape(N_DEV, CHUNK)

    def body(idx_ref, table_ref, out_ref, tblb_ref, p0_ref, p1_ref, red_ref,
             pbo_buf, pbf_buf, mir_buf, p1_send, p1_recv,
             pbo_send, pbo_recv, pbf_send, pbf_recv, mir_send, mir_recv):
        my = lax.axis_index("i")

        barrier_sem = pltpu.get_barrier_semaphore()
        for o in range(1, N_DEV):
            pl.semaphore_signal(
                barrier_sem, inc=1,
                device_id=(jnp.mod(my + o, N_DEV),),
                device_id_type=pl.DeviceIdType.MESH,
            )

        tblb_ref[...] = table_ref[...].astype(jnp.bfloat16)

        base = my * V_PER
        SLAB = 4
        col = lax.broadcasted_iota(jnp.int32, (CHUNK, V_PER), 1).astype(
            jnp.int16)

        def gather_slab(offsets):
            blocks = []
            for o in offsets:
                c = jnp.mod(my + o, N_DEV)
                loc = (idx_ref[pl.ds(c, 1), :].reshape(CHUNK, 1)
                       - base).astype(jnp.int16)
                blocks.append((col == loc).astype(jnp.bfloat16))
            oh = jnp.concatenate(blocks, axis=0)
            part = jnp.dot(oh, tblb_ref[...],
                           preferred_element_type=jnp.float32)
            return part.astype(jnp.bfloat16)

        p1_rdmas = {}
        for s in range(N_DEV // SLAB):
            offsets = [(s * SLAB + j + 1) % N_DEV for j in range(SLAB)]
            part = gather_slab(offsets)
            if s == 0:
                pl.semaphore_wait(barrier_sem, N_DEV - 1)
            for j, o in enumerate(offsets):
                p0_ref[o] = part[j * CHUNK:(j + 1) * CHUNK, :]
                if o == 0:
                    continue
                rdma = pltpu.make_async_remote_copy(
                    src_ref=p0_ref.at[o],
                    dst_ref=p1_ref.at[o],
                    send_sem=p1_send.at[o],
                    recv_sem=p1_recv.at[o],
                    device_id=(jnp.mod(my + o, N_DEV),),
                    device_id_type=pl.DeviceIdType.MESH,
                )
                rdma.start()
                p1_rdmas[o] = rdma
        p1_rdmas = [p1_rdmas[o] for o in range(1, N_DEV)]

        acc = p0_ref[0]
        for o in range(1, N_DEV):
            p1_rdmas[o - 1].wait_recv()
            acc = acc + p1_ref[o]
        red_ref[...] = acc

        zz = jnp.floor_divide(my, 4)
        pp = jnp.mod(my, 4)
        MATES = [(dq, df) for df in (0, 1) for dq in range(4)
                 if (dq, df) != (0, 0)]

        def mate_id(dq, df):
            return 4 * jnp.bitwise_xor(zz, df) + jnp.mod(pp + dq, 4)

        mir_id = 4 * jnp.bitwise_xor(zz, 2) + pp

        own_rdmas = []
        for dq, df in MATES:
            slot = ((4 - dq) % 4) * 2 + df
            r = pltpu.make_async_remote_copy(
                src_ref=red_ref,
                dst_ref=pbo_buf.at[slot],
                send_sem=pbo_send.at[slot],
                recv_sem=pbo_recv.at[slot],
                device_id=(mate_id(dq, df),),
                device_id_type=pl.DeviceIdType.MESH,
            )
            r.start()
            own_rdmas.append(r)
        mir_rdma = pltpu.make_async_remote_copy(
            src_ref=red_ref,
            dst_ref=mir_buf,
            send_sem=mir_send.at[0],
            recv_sem=mir_recv.at[0],
            device_id=(mir_id,),
            device_id_type=pl.DeviceIdType.MESH,
        )
        mir_rdma.start()

        out_ref[pl.ds(my * CHUNK, CHUNK), :] = red_ref[...].astype(jnp.float32)

        for r in p1_rdmas:
            r.wait_send()

        mir_rdma.wait_recv()
        fwd_rdmas = []
        for dq, df in MATES:
            slot = ((4 - dq) % 4) * 2 + df
            r = pltpu.make_async_remote_copy(
                src_ref=mir_buf,
                dst_ref=pbf_buf.at[slot],
                send_sem=pbf_send.at[slot],
                recv_sem=pbf_recv.at[slot],
                device_id=(mate_id(dq, df),),
                device_id_type=pl.DeviceIdType.MESH,
            )
            r.start()
            fwd_rdmas.append(r)
        out_ref[pl.ds(mir_id * CHUNK, CHUNK), :] = mir_buf[...].astype(
            jnp.float32)

        for dq, df in MATES:
            rr = pltpu.make_async_remote_copy(
                src_ref=red_ref,
                dst_ref=pbo_buf.at[dq * 2 + df],
                send_sem=pbo_send.at[dq * 2 + df],
                recv_sem=pbo_recv.at[dq * 2 + df],
                device_id=(mate_id(dq, df),),
                device_id_type=pl.DeviceIdType.MESH,
            )
            rr.wait_recv()
            cid = mate_id(dq, df)
            out_ref[pl.ds(cid * CHUNK, CHUNK), :] = pbo_buf[dq * 2 + df].astype(
                jnp.float32)
        for dq, df in MATES:
            rr = pltpu.make_async_remote_copy(
                src_ref=red_ref,
                dst_ref=pbf_buf.at[dq * 2 + df],
                send_sem=pbf_send.at[dq * 2 + df],
                recv_sem=pbf_recv.at[dq * 2 + df],
                device_id=(mate_id(dq, df),),
                device_id_type=pl.DeviceIdType.MESH,
            )
            rr.wait_recv()
            cid = 4 * jnp.bitwise_xor(jnp.bitwise_xor(zz, df), 2) + jnp.mod(
                pp + dq, 4)
            out_ref[pl.ds(cid * CHUNK, CHUNK), :] = pbf_buf[dq * 2 + df].astype(
                jnp.float32)

        for r in own_rdmas:
            r.wait_send()
        mir_rdma.wait_send()
        for r in fwd_rdmas:
            r.wait_send()

    return pl.pallas_call(
        body,
        out_shape=jax.ShapeDtypeStruct((N_IDX, D), jnp.float32),
        in_specs=[
            pl.BlockSpec(memory_space=pltpu.VMEM),
            pl.BlockSpec(memory_space=pltpu.VMEM),
        ],
        out_specs=pl.BlockSpec(memory_space=pltpu.VMEM),
        scratch_shapes=[
            pltpu.VMEM((V_PER, D), jnp.bfloat16),
            pltpu.VMEM((N_DEV, CHUNK, D), jnp.bfloat16),
            pltpu.VMEM((N_DEV, CHUNK, D), jnp.bfloat16),
            pltpu.VMEM((CHUNK, D), jnp.bfloat16),
            pltpu.VMEM((8, CHUNK, D), jnp.bfloat16),
            pltpu.VMEM((8, CHUNK, D), jnp.bfloat16),
            pltpu.VMEM((CHUNK, D), jnp.bfloat16),
            pltpu.SemaphoreType.DMA((N_DEV,)),
            pltpu.SemaphoreType.DMA((N_DEV,)),
            pltpu.SemaphoreType.DMA((8,)),
            pltpu.SemaphoreType.DMA((8,)),
            pltpu.SemaphoreType.DMA((8,)),
            pltpu.SemaphoreType.DMA((8,)),
            pltpu.SemaphoreType.DMA((1,)),
            pltpu.SemaphoreType.DMA((1,)),
        ],
        compiler_params=pltpu.CompilerParams(collective_id=0),
    )(idx2, table)


# device time: 35971 ns/iter; 1.0307x vs baseline; 1.0307x over previous
import jax
import jax.numpy as jnp
from jax import lax
from jax.experimental import pallas as pl
from jax.experimental.pallas import tpu as pltpu

N_DEV = 16
N_IDX = 1024
D = 512
HD = D // 2
V_PER = 4096
CHUNK = N_IDX // N_DEV


def kernel(table, idx):
    idx2 = idx.reshape(N_DEV, CHUNK)

    def body(idx_ref, table_ref, out_ref, tblb_ref,
             p0a_ref, p0b_ref, p1a_ref, p1b_ref, reda_ref, redb_ref,
             p2a_ref, p2b_ref,
             p1a_send, p1a_recv, p1b_send, p1b_recv,
             p2a_send, p2a_recv, p2b_send, p2b_recv):
        my = lax.axis_index("i")

        barrier_sem = pltpu.get_barrier_semaphore()
        for o in range(1, N_DEV):
            pl.semaphore_signal(
                barrier_sem, inc=1,
                device_id=(jnp.mod(my + o, N_DEV),),
                device_id_type=pl.DeviceIdType.MESH,
            )

        tblb_ref[...] = table_ref[...].astype(jnp.bfloat16)

        base = my * V_PER
        SLAB = 4
        col = lax.broadcasted_iota(jnp.int32, (CHUNK, V_PER), 1).astype(
            jnp.int16)

        def gather_slab(offsets):
            blocks = []
            for o in offsets:
                c = jnp.mod(my + o, N_DEV)
                loc = (idx_ref[pl.ds(c, 1), :].reshape(CHUNK, 1)
                       - base).astype(jnp.int16)
                blocks.append((col == loc).astype(jnp.bfloat16))
            oh = jnp.concatenate(blocks, axis=0)
            part = jnp.dot(oh, tblb_ref[...],
                           preferred_element_type=jnp.float32)
            return part.astype(jnp.bfloat16)

        def send_slot(src, dst, ssem, rsem, o):
            r = pltpu.make_async_remote_copy(
                src_ref=src if o is None else src.at[o],
                dst_ref=dst.at[o],
                send_sem=ssem.at[o], recv_sem=rsem.at[o],
                device_id=(jnp.mod(my + o, N_DEV),),
                device_id_type=pl.DeviceIdType.MESH,
            )
            r.start()
            return r

        p1a_rdmas = {}
        for s in range(N_DEV // SLAB):
            offsets = [(s * SLAB + j + 1) % N_DEV for j in range(SLAB)]
            part = gather_slab(offsets)
            if s == 0:
                pl.semaphore_wait(barrier_sem, N_DEV - 1)
            for j, o in enumerate(offsets):
                p0a_ref[o] = part[j * CHUNK:(j + 1) * CHUNK, :HD]
                p0b_ref[o] = part[j * CHUNK:(j + 1) * CHUNK, HD:]
                if o:
                    p1a_rdmas[o] = send_slot(p0a_ref, p1a_ref, p1a_send,
                                             p1a_recv, o)
        p1b_rdmas = {o: send_slot(p0b_ref, p1b_ref, p1b_send, p1b_recv, o)
                     for o in range(1, N_DEV)}

        acc = p0a_ref[0]
        for o in range(1, N_DEV):
            p1a_rdmas[o].wait_recv()
            acc = acc + p1a_ref[o]
        reda_ref[...] = acc
        p2a_rdmas = []
        for o in range(1, N_DEV):
            r = pltpu.make_async_remote_copy(
                src_ref=reda_ref, dst_ref=p2a_ref.at[o],
                send_sem=p2a_send.at[o], recv_sem=p2a_recv.at[o],
                device_id=(jnp.mod(my + o, N_DEV),),
                device_id_type=pl.DeviceIdType.MESH,
            )
            r.start()
            p2a_rdmas.append(r)
        out_ref[pl.ds(my * CHUNK, CHUNK), :HD] = reda_ref[...].astype(
            jnp.float32)

        acc = p0b_ref[0]
        for o in range(1, N_DEV):
            p1b_rdmas[o].wait_recv()
            acc = acc + p1b_ref[o]
        redb_ref[...] = acc
        p2b_rdmas = []
        for o in range(1, N_DEV):
            r = pltpu.make_async_remote_copy(
                src_ref=redb_ref, dst_ref=p2b_ref.at[o],
                send_sem=p2b_send.at[o], recv_sem=p2b_recv.at[o],
                device_id=(jnp.mod(my + o, N_DEV),),
                device_id_type=pl.DeviceIdType.MESH,
            )
            r.start()
            p2b_rdmas.append(r)
        out_ref[pl.ds(my * CHUNK, CHUNK), HD:] = redb_ref[...].astype(
            jnp.float32)

        for o in range(1, N_DEV):
            p1a_rdmas[o].wait_send()
            p1b_rdmas[o].wait_send()

        for o in range(1, N_DEV):
            p2a_rdmas[o - 1].wait_recv()
            c = jnp.mod(my - o, N_DEV)
            out_ref[pl.ds(c * CHUNK, CHUNK), :HD] = p2a_ref[o].astype(
                jnp.float32)
        for o in range(1, N_DEV):
            p2b_rdmas[o - 1].wait_recv()
            c = jnp.mod(my - o, N_DEV)
            out_ref[pl.ds(c * CHUNK, CHUNK), HD:] = p2b_ref[o].astype(
                jnp.float32)
        for r in p2a_rdmas:
            r.wait_send()
        for r in p2b_rdmas:
            r.wait_send()

    return pl.pallas_call(
        body,
        out_shape=jax.ShapeDtypeStruct((N_IDX, D), jnp.float32),
        in_specs=[
            pl.BlockSpec(memory_space=pltpu.VMEM),
            pl.BlockSpec(memory_space=pltpu.VMEM),
        ],
        out_specs=pl.BlockSpec(memory_space=pltpu.VMEM),
        scratch_shapes=[
            pltpu.VMEM((V_PER, D), jnp.bfloat16),
            pltpu.VMEM((N_DEV, CHUNK, HD), jnp.bfloat16),
            pltpu.VMEM((N_DEV, CHUNK, HD), jnp.bfloat16),
            pltpu.VMEM((N_DEV, CHUNK, HD), jnp.bfloat16),
            pltpu.VMEM((N_DEV, CHUNK, HD), jnp.bfloat16),
            pltpu.VMEM((CHUNK, HD), jnp.bfloat16),
            pltpu.VMEM((CHUNK, HD), jnp.bfloat16),
            pltpu.VMEM((N_DEV, CHUNK, HD), jnp.bfloat16),
            pltpu.VMEM((N_DEV, CHUNK, HD), jnp.bfloat16),
            pltpu.SemaphoreType.DMA((N_DEV,)),
            pltpu.SemaphoreType.DMA((N_DEV,)),
            pltpu.SemaphoreType.DMA((N_DEV,)),
            pltpu.SemaphoreType.DMA((N_DEV,)),
            pltpu.SemaphoreType.DMA((N_DEV,)),
            pltpu.SemaphoreType.DMA((N_DEV,)),
            pltpu.SemaphoreType.DMA((N_DEV,)),
            pltpu.SemaphoreType.DMA((N_DEV,)),
        ],
        compiler_params=pltpu.CompilerParams(collective_id=0),
    )(idx2, table)


# device time: 34175 ns/iter; 1.0849x vs baseline; 1.0526x over previous
import jax
import jax.numpy as jnp
from jax import lax
from jax.experimental import pallas as pl
from jax.experimental.pallas import tpu as pltpu

N_DEV = 16
N_IDX = 1024
D = 512
V_PER = 4096
CHUNK = N_IDX // N_DEV


def kernel(table, idx):
    idx2 = idx.reshape(N_DEV, CHUNK)

    def body(idx_ref, table_ref, out_ref, tblb_ref, p0_ref, p1_ref, red_ref,
             p2_ref, p1_send, p1_recv, p2_send, p2_recv):
        my = lax.axis_index("i")

        barrier_sem = pltpu.get_barrier_semaphore()
        for o in range(1, N_DEV):
            pl.semaphore_signal(
                barrier_sem, inc=1,
                device_id=(jnp.mod(my + o, N_DEV),),
                device_id_type=pl.DeviceIdType.MESH,
            )

        tblb_ref[...] = table_ref[...].astype(jnp.bfloat16)

        base = my * V_PER
        SLAB = 4
        col = lax.broadcasted_iota(jnp.int32, (CHUNK, V_PER), 1).astype(
            jnp.int16)

        def gather_slab(offsets):
            blocks = []
            for o in offsets:
                c = jnp.mod(my + o, N_DEV)
                loc = (idx_ref[pl.ds(c, 1), :].reshape(CHUNK, 1)
                       - base).astype(jnp.int16)
                blocks.append((col == loc).astype(jnp.bfloat16))
            oh = jnp.concatenate(blocks, axis=0)
            part = jnp.dot(oh, tblb_ref[...],
                           preferred_element_type=jnp.float32)
            return part.astype(jnp.bfloat16)

        p1_rdmas = {}
        for s in range(N_DEV // SLAB):
            offsets = [(s * SLAB + j + 1) % N_DEV for j in range(SLAB)]
            part = gather_slab(offsets)
            if s == 0:
                pl.semaphore_wait(barrier_sem, N_DEV - 1)
            for j, o in enumerate(offsets):
                p0_ref[o] = part[j * CHUNK:(j + 1) * CHUNK, :]
                if o == 0:
                    continue
                rdma = pltpu.make_async_remote_copy(
                    src_ref=p0_ref.at[o],
                    dst_ref=p1_ref.at[o],
                    send_sem=p1_send.at[o],
                    recv_sem=p1_recv.at[o],
                    device_id=(jnp.mod(my + o, N_DEV),),
                    device_id_type=pl.DeviceIdType.MESH,
                )
                rdma.start()
                p1_rdmas[o] = rdma
        p1_rdmas = [p1_rdmas[o] for o in range(1, N_DEV)]

        acc = p0_ref[0]
        for o in range(1, N_DEV):
            p1_rdmas[o - 1].wait_recv()
            acc = acc + p1_ref[o]
        red_ref[...] = acc

        p2_rdmas = []
        for o in range(1, N_DEV):
            e = jnp.mod(my + o, N_DEV)
            rdma = pltpu.make_async_remote_copy(
                src_ref=red_ref,
                dst_ref=p2_ref.at[o],
                send_sem=p2_send.at[o],
                recv_sem=p2_recv.at[o],
                device_id=(e,),
                device_id_type=pl.DeviceIdType.MESH,
            )
            rdma.start()
            p2_rdmas.append(rdma)

        out_ref[pl.ds(my * CHUNK, CHUNK), :] = red_ref[...].astype(jnp.float32)

        for r in p1_rdmas:
            r.wait_send()

        for o in range(1, N_DEV):
            p2_rdmas[o - 1].wait_recv()
            c = jnp.mod(my - o, N_DEV)
            out_ref[pl.ds(c * CHUNK, CHUNK), :] = p2_ref[o].astype(jnp.float32)
        for r in p2_rdmas:
            r.wait_send()

    return pl.pallas_call(
        body,
        out_shape=jax.ShapeDtypeStruct((N_IDX, D), jnp.float32),
        in_specs=[
            pl.BlockSpec(memory_space=pltpu.VMEM),
            pl.BlockSpec(memory_space=pltpu.VMEM),
        ],
        out_specs=pl.BlockSpec(memory_space=pltpu.VMEM),
        scratch_shapes=[
            pltpu.VMEM((V_PER, D), jnp.bfloat16),
            pltpu.VMEM((N_DEV, CHUNK, D), jnp.bfloat16),
            pltpu.VMEM((N_DEV, CHUNK, D), jnp.bfloat16),
            pltpu.VMEM((CHUNK, D), jnp.bfloat16),
            pltpu.VMEM((N_DEV, CHUNK, D), jnp.bfloat16),
            pltpu.SemaphoreType.DMA((N_DEV,)),
            pltpu.SemaphoreType.DMA((N_DEV,)),
            pltpu.SemaphoreType.DMA((N_DEV,)),
            pltpu.SemaphoreType.DMA((N_DEV,)),
        ],
        compiler_params=pltpu.CompilerParams(collective_id=0),
    )(idx2, table)
